# Initial kernel scaffold; baseline (speedup 1.0000x reference)
import jax
import jax.numpy as jnp
from jax.experimental import pallas as pl


def kernel(x, edge_index, W1l, b1l, W1r, gn_w, gn_b, gn_ms, W2l, b2l, W2r):
    raise NotImplementedError("write your pallas kernel here")


# speedup vs baseline: 3.4095x; 3.4095x over previous
import functools

import jax
import jax.numpy as jnp
from jax import lax
from jax.experimental import pallas as pl
from jax.experimental.pallas import tpu as pltpu
from jax.experimental.pallas import tpu_sc as plsc

N_NODES = 10000
D_FEAT = 128
EPS = 1e-5

NW = 32
CHUNK = 128
K_CHUNKS = 80
E_PAD = NW * K_CHUNKS * CHUNK
ACC_ROWS = 10112
CNT_ROWS = 80


def _make_sc_scatter(count: bool):
    mesh = plsc.VectorSubcoreMesh(core_axis_name="c", subcore_axis_name="s")
    rows_per_tile = ACC_ROWS // 16

    out_type = [jax.ShapeDtypeStruct((2, ACC_ROWS, D_FEAT), jnp.float32)]
    scratch = [
        pltpu.VMEM((K_CHUNKS, CHUNK), jnp.int32),
        pltpu.VMEM((K_CHUNKS, CHUNK), jnp.int32),
        pltpu.VMEM((CHUNK, D_FEAT), jnp.float32),
        pltpu.VMEM_SHARED((ACC_ROWS, D_FEAT), jnp.float32),
        pltpu.SemaphoreType.DMA,
    ]
    if count:
        out_type.append(jax.ShapeDtypeStruct((2, CNT_ROWS, D_FEAT), jnp.float32))
        scratch += [
            pltpu.VMEM((CNT_ROWS, D_FEAT), jnp.float32),
            pltpu.VMEM((CNT_ROWS,), jnp.int32),
            pltpu.VMEM_SHARED((CNT_ROWS, D_FEAT), jnp.float32),
        ]

    def body(table_hbm, src_hbm, dst_hbm, out_hbm, *rest):
        if count:
            (cnt_hbm, src_v, dst_v, rows_v, acc_sh, sem, cnt_v,
             iota_v, cnt_sh) = rest
        else:
            src_v, dst_v, rows_v, acc_sh, sem = rest
        c = lax.axis_index("c")
        s = lax.axis_index("s")
        wid = c * 16 + s

        zerof = jnp.zeros((16,), jnp.float32)

        def zrow(r, carry):
            for j in range(D_FEAT // 16):
                rows_v[r, pl.ds(j * 16, 16)] = zerof
            return carry

        lax.fori_loop(0, CHUNK, zrow, 0)
        off = 0
        while off < rows_per_tile:
            sz = min(CHUNK, rows_per_tile - off)
            pltpu.sync_copy(
                rows_v.at[pl.ds(0, sz)],
                acc_sh.at[pl.ds(s * rows_per_tile + off, sz)],
            )
            off += sz
        if count:
            def zcnt(r, carry):
                for j in range(D_FEAT // 16):
                    cnt_v[r, pl.ds(j * 16, 16)] = zerof
                return carry

            lax.fori_loop(0, CNT_ROWS, zcnt, 0)

            @pl.when(s == 0)
            def _():
                pltpu.sync_copy(cnt_v, cnt_sh)

            def ziota(i, carry):
                iota_v[pl.ds(i * 16, 16)] = (
                    jnp.arange(16, dtype=jnp.int32) + i * 16)
                return carry

            lax.fori_loop(0, CNT_ROWS // 16, ziota, 0)
        plsc.subcore_barrier()

        pltpu.sync_copy(src_hbm.at[wid], src_v)
        pltpu.sync_copy(dst_hbm.at[wid], dst_v)

        ones16 = jnp.ones((16,), jnp.float32)

        def step(j, carry):
            gather = pltpu.async_copy(table_hbm.at[src_v.at[j]], rows_v, sem)
            if count:
                for g in range(CHUNK // 16):
                    idx = dst_v[j, pl.ds(g * 16, 16)]
                    plsc.addupdate_scatter(
                        cnt_v,
                        [lax.shift_right_logical(idx, 7),
                         jnp.bitwise_and(idx, 127)],
                        ones16)
            gather.wait()
            pltpu.sync_copy(rows_v, acc_sh.at[dst_v.at[j]], add=True)
            return carry

        lax.fori_loop(0, K_CHUNKS, step, 0)

        if count:
            pltpu.sync_copy(cnt_v, cnt_sh.at[iota_v], add=True)
        plsc.subcore_barrier()

        pltpu.sync_copy(
            acc_sh.at[pl.ds(s * rows_per_tile, rows_per_tile)],
            out_hbm.at[c].at[pl.ds(s * rows_per_tile, rows_per_tile)],
        )
        if count:
            @pl.when(s < CNT_ROWS // 16)
            def _():
                pltpu.sync_copy(
                    cnt_sh.at[pl.ds(s * 16, 16)],
                    cnt_hbm.at[c].at[pl.ds(s * 16, 16)],
                )

    return functools.partial(
        pl.kernel, mesh=mesh, out_type=out_type, scratch_types=scratch,
        compiler_params=pltpu.CompilerParams(needs_layout_passes=False))(body)


_sc_scatter_cnt = _make_sc_scatter(count=True)
_sc_scatter = _make_sc_scatter(count=False)


def _tc1_body(p_ref, c_ref, x_ref, w1lT_ref, b1l_ref, w1rT_ref, gnw_ref,
              gnb_ref, gnms_ref, h_ref, invc_ref):
    agg = p_ref[0, :N_NODES] + p_ref[1, :N_NODES]
    cnt = c_ref[0] + c_ref[1]
    invc = 1.0 / jnp.maximum(cnt, 1.0)
    mean = agg * invc
    h = (jnp.dot(mean, w1lT_ref[...], preferred_element_type=jnp.float32)
         + b1l_ref[...]
         + jnp.dot(x_ref[...], w1rT_ref[...], preferred_element_type=jnp.float32))
    mu = jnp.mean(h, axis=0, keepdims=True)
    o = h - gnms_ref[...] * mu
    var = jnp.mean(o * o, axis=0, keepdims=True)
    g = gnw_ref[...] * o * lax.rsqrt(var + EPS) + gnb_ref[...]
    h_ref[...] = jnp.maximum(g, 0.0)
    invc_ref[...] = invc


def _tc2_body(p_ref, h_ref, invc_ref, w2lT_ref, b2l_ref, w2rT_ref, out_ref):
    mean = (p_ref[0, :N_NODES] + p_ref[1, :N_NODES]) * invc_ref[...]
    out_ref[...] = (jnp.dot(mean, w2lT_ref[...], preferred_element_type=jnp.float32)
                    + b2l_ref[...]
                    + jnp.dot(h_ref[...], w2rT_ref[...],
                              preferred_element_type=jnp.float32))


def kernel(x, edge_index, W1l, b1l, W1r, gn_w, gn_b, gn_ms, W2l, b2l, W2r):
    src = edge_index[0]
    dst = edge_index[1]
    e = src.shape[0]
    pad = E_PAD - e
    src_p = jnp.concatenate([src, jnp.zeros((pad,), jnp.int32)]).reshape(
        NW, K_CHUNKS, CHUNK)
    dst_p = jnp.concatenate([dst, jnp.full((pad,), N_NODES, jnp.int32)]).reshape(
        NW, K_CHUNKS, CHUNK)

    part1, cnt_p = _sc_scatter_cnt(x, src_p, dst_p)
    cnt2 = cnt_p.reshape(2, CNT_ROWS * D_FEAT, 1)[:, :N_NODES]

    h, invc = pl.pallas_call(
        _tc1_body,
        out_shape=[
            jax.ShapeDtypeStruct((N_NODES, D_FEAT), jnp.float32),
            jax.ShapeDtypeStruct((N_NODES, 1), jnp.float32),
        ],
    )(part1, cnt2, x, W1l.T, b1l.reshape(1, -1), W1r.T, gn_w.reshape(1, -1),
      gn_b.reshape(1, -1), gn_ms.reshape(1, -1))

    (part2,) = _sc_scatter(h, src_p, dst_p)

    out = pl.pallas_call(
        _tc2_body,
        out_shape=jax.ShapeDtypeStruct((N_NODES, D_FEAT), jnp.float32),
    )(part2, h, invc, W2l.T, b2l.reshape(1, -1), W2r.T)
    return out


# speedup vs baseline: 3.8516x; 1.1297x over previous
import functools

import jax
import jax.numpy as jnp
from jax import lax
from jax.experimental import pallas as pl
from jax.experimental.pallas import tpu as pltpu
from jax.experimental.pallas import tpu_sc as plsc

N_NODES = 10000
D_FEAT = 128
EPS = 1e-5

NW = 32
CHUNK = 64
K_CHUNKS = 160
E_PAD = NW * K_CHUNKS * CHUNK
ACC_ROWS = 10112
CNT_ROWS = 80


def _make_sc_scatter(count: bool):
    mesh = plsc.VectorSubcoreMesh(core_axis_name="c", subcore_axis_name="s")
    rows_per_tile = ACC_ROWS // 16

    out_type = [jax.ShapeDtypeStruct((2, ACC_ROWS, D_FEAT), jnp.float32)]
    scratch = [
        pltpu.VMEM((K_CHUNKS, CHUNK), jnp.int32),
        pltpu.VMEM((CHUNK,), jnp.int32),
        pltpu.VMEM((CHUNK,), jnp.int32),
        pltpu.VMEM((CHUNK,), jnp.int32),
        pltpu.VMEM((CHUNK,), jnp.int32),
        pltpu.VMEM((CHUNK, D_FEAT), jnp.float32),
        pltpu.VMEM((CHUNK, D_FEAT), jnp.float32),
        pltpu.VMEM_SHARED((ACC_ROWS, D_FEAT), jnp.float32),
        pltpu.SemaphoreType.DMA,
        pltpu.SemaphoreType.DMA,
    ]
    if count:
        out_type.append(jax.ShapeDtypeStruct((2, CNT_ROWS, D_FEAT), jnp.float32))
        scratch += [
            pltpu.VMEM((CNT_ROWS, D_FEAT), jnp.float32),
            pltpu.VMEM((CNT_ROWS,), jnp.int32),
            pltpu.VMEM_SHARED((CNT_ROWS, D_FEAT), jnp.float32),
        ]

    def body(table_hbm, packed_hbm, out_hbm, *rest):
        if count:
            (cnt_hbm, packed_v, src_a, dst_a, src_b, dst_b, rows_a, rows_b,
             acc_sh, sem_a, sem_b, cnt_v, iota_v, cnt_sh) = rest
        else:
            (packed_v, src_a, dst_a, src_b, dst_b, rows_a, rows_b,
             acc_sh, sem_a, sem_b) = rest
        c = lax.axis_index("c")
        s = lax.axis_index("s")
        wid = c * 16 + s

        zerof = jnp.zeros((16,), jnp.float32)

        def zrow(r, carry):
            for j in range(D_FEAT // 16):
                rows_a[r, pl.ds(j * 16, 16)] = zerof
            return carry

        lax.fori_loop(0, CHUNK, zrow, 0)
        off = 0
        while off < rows_per_tile:
            sz = min(CHUNK, rows_per_tile - off)
            pltpu.sync_copy(
                rows_a.at[pl.ds(0, sz)],
                acc_sh.at[pl.ds(s * rows_per_tile + off, sz)],
            )
            off += sz
        if count:
            def zcnt(r, carry):
                for j in range(D_FEAT // 16):
                    cnt_v[r, pl.ds(j * 16, 16)] = zerof
                return carry

            lax.fori_loop(0, CNT_ROWS, zcnt, 0)

            @pl.when(s == 0)
            def _():
                pltpu.sync_copy(cnt_v, cnt_sh)

            def ziota(i, carry):
                iota_v[pl.ds(i * 16, 16)] = (
                    jnp.arange(16, dtype=jnp.int32) + i * 16)
                return carry

            lax.fori_loop(0, CNT_ROWS // 16, ziota, 0)
        plsc.subcore_barrier()

        pltpu.sync_copy(packed_hbm.at[wid], packed_v)

        ones16 = jnp.ones((16,), jnp.float32)

        def unpack(j, srcb, dstb):
            for g in range(CHUNK // 16):
                v = packed_v[j, pl.ds(g * 16, 16)]
                srcb[pl.ds(g * 16, 16)] = jnp.bitwise_and(v, 16383)
                dstb[pl.ds(g * 16, 16)] = lax.shift_right_logical(v, 14)

        def do_count(j):
            for g in range(CHUNK // 16):
                idx = lax.shift_right_logical(
                    packed_v[j, pl.ds(g * 16, 16)], 14)
                plsc.addupdate_scatter(
                    cnt_v,
                    [lax.shift_right_logical(idx, 7),
                     jnp.bitwise_and(idx, 127)],
                    ones16)

        unpack(0, src_a, dst_a)
        pltpu.async_copy(table_hbm.at[src_a], rows_a, sem_a)
        unpack(1, src_b, dst_b)
        pltpu.async_copy(table_hbm.at[src_b], rows_b, sem_b)

        def pair(p, carry):
            j0 = 2 * p
            pltpu.make_async_copy(
                table_hbm.at[pl.ds(0, CHUNK)], rows_a, sem_a).wait()
            pltpu.sync_copy(rows_a, acc_sh.at[dst_a], add=True)

            @pl.when(j0 + 2 < K_CHUNKS)
            def _():
                unpack(j0 + 2, src_a, dst_a)
                pltpu.async_copy(table_hbm.at[src_a], rows_a, sem_a)
            if count:
                do_count(j0)

            pltpu.make_async_copy(
                table_hbm.at[pl.ds(0, CHUNK)], rows_b, sem_b).wait()
            pltpu.sync_copy(rows_b, acc_sh.at[dst_b], add=True)

            @pl.when(j0 + 3 < K_CHUNKS)
            def _():
                unpack(j0 + 3, src_b, dst_b)
                pltpu.async_copy(table_hbm.at[src_b], rows_b, sem_b)
            if count:
                do_count(j0 + 1)
            return carry

        lax.fori_loop(0, K_CHUNKS // 2, pair, 0)

        if count:
            pltpu.sync_copy(cnt_v, cnt_sh.at[iota_v], add=True)
        plsc.subcore_barrier()

        pltpu.sync_copy(
            acc_sh.at[pl.ds(s * rows_per_tile, rows_per_tile)],
            out_hbm.at[c].at[pl.ds(s * rows_per_tile, rows_per_tile)],
        )
        if count:
            @pl.when(s < CNT_ROWS // 16)
            def _():
                pltpu.sync_copy(
                    cnt_sh.at[pl.ds(s * 16, 16)],
                    cnt_hbm.at[c].at[pl.ds(s * 16, 16)],
                )

    return functools.partial(
        pl.kernel, mesh=mesh, out_type=out_type, scratch_types=scratch,
        compiler_params=pltpu.CompilerParams(needs_layout_passes=False))(body)


_sc_scatter_cnt = _make_sc_scatter(count=True)
_sc_scatter = _make_sc_scatter(count=False)


def _tc1_body(p_ref, c_ref, x_ref, w1lT_ref, b1l_ref, w1rT_ref, gnw_ref,
              gnb_ref, gnms_ref, h_ref, invc_ref):
    agg = p_ref[0, :N_NODES] + p_ref[1, :N_NODES]
    cnt = c_ref[0] + c_ref[1]
    invc = 1.0 / jnp.maximum(cnt, 1.0)
    mean = agg * invc
    h = (jnp.dot(mean, w1lT_ref[...], preferred_element_type=jnp.float32)
         + b1l_ref[...]
         + jnp.dot(x_ref[...], w1rT_ref[...], preferred_element_type=jnp.float32))
    mu = jnp.mean(h, axis=0, keepdims=True)
    o = h - gnms_ref[...] * mu
    var = jnp.mean(o * o, axis=0, keepdims=True)
    g = gnw_ref[...] * o * lax.rsqrt(var + EPS) + gnb_ref[...]
    h_ref[...] = jnp.maximum(g, 0.0)
    invc_ref[...] = invc


def _tc2_body(p_ref, h_ref, invc_ref, w2lT_ref, b2l_ref, w2rT_ref, out_ref):
    mean = (p_ref[0, :N_NODES] + p_ref[1, :N_NODES]) * invc_ref[...]
    out_ref[...] = (jnp.dot(mean, w2lT_ref[...], preferred_element_type=jnp.float32)
                    + b2l_ref[...]
                    + jnp.dot(h_ref[...], w2rT_ref[...],
                              preferred_element_type=jnp.float32))


def kernel(x, edge_index, W1l, b1l, W1r, gn_w, gn_b, gn_ms, W2l, b2l, W2r):
    src = edge_index[0]
    dst = edge_index[1]
    e = src.shape[0]
    pad = E_PAD - e
    packed = jnp.bitwise_or(src, jnp.left_shift(dst, 14))
    packed_p = jnp.concatenate(
        [packed, jnp.full((pad,), N_NODES << 14, jnp.int32)]
    ).reshape(NW, K_CHUNKS, CHUNK)

    part1, cnt_p = _sc_scatter_cnt(x, packed_p)
    cnt2 = cnt_p.reshape(2, CNT_ROWS * D_FEAT, 1)[:, :N_NODES]

    h, invc = pl.pallas_call(
        _tc1_body,
        out_shape=[
            jax.ShapeDtypeStruct((N_NODES, D_FEAT), jnp.float32),
            jax.ShapeDtypeStruct((N_NODES, 1), jnp.float32),
        ],
    )(part1, cnt2, x, W1l.T, b1l.reshape(1, -1), W1r.T, gn_w.reshape(1, -1),
      gn_b.reshape(1, -1), gn_ms.reshape(1, -1))

    (part2,) = _sc_scatter(h, packed_p)

    out = pl.pallas_call(
        _tc2_body,
        out_shape=jax.ShapeDtypeStruct((N_NODES, D_FEAT), jnp.float32),
    )(part2, h, invc, W2l.T, b2l.reshape(1, -1), W2r.T)
    return out


# speedup vs baseline: 3.8557x; 1.0010x over previous
import functools

import jax
import jax.numpy as jnp
from jax import lax
from jax.experimental import pallas as pl
from jax.experimental.pallas import tpu as pltpu
from jax.experimental.pallas import tpu_sc as plsc

N_NODES = 10000
D_FEAT = 128
EPS = 1e-5

NW = 32
CHUNK = 64
K_FAST = 240
K_SLOW = 80
E_FAST = 16 * K_FAST * CHUNK
E_SLOW = 16 * K_SLOW * CHUNK
E_PAD = E_FAST + E_SLOW
ACC_ROWS = 10112
CNT_ROWS = 80
DUMMY = 10000 << 14


def _make_sc_scatter(count: bool):
    mesh = plsc.VectorSubcoreMesh(core_axis_name="c", subcore_axis_name="s")
    rows_per_tile = ACC_ROWS // 16

    out_type = [jax.ShapeDtypeStruct((2, ACC_ROWS, D_FEAT), jnp.float32)]
    scratch = [
        pltpu.VMEM((K_FAST // 2, CHUNK), jnp.int32),
        pltpu.VMEM((CHUNK,), jnp.int32),
        pltpu.VMEM((CHUNK,), jnp.int32),
        pltpu.VMEM((CHUNK,), jnp.int32),
        pltpu.VMEM((CHUNK,), jnp.int32),
        pltpu.VMEM((CHUNK, D_FEAT), jnp.float32),
        pltpu.VMEM((CHUNK, D_FEAT), jnp.float32),
        pltpu.VMEM_SHARED((ACC_ROWS, D_FEAT), jnp.float32),
        pltpu.SemaphoreType.DMA,
        pltpu.SemaphoreType.DMA,
    ]
    if count:
        out_type.append(jax.ShapeDtypeStruct((2, CNT_ROWS, D_FEAT), jnp.float32))
        scratch += [
            pltpu.VMEM((CNT_ROWS, D_FEAT), jnp.float32),
            pltpu.VMEM((CNT_ROWS,), jnp.int32),
            pltpu.VMEM_SHARED((CNT_ROWS, D_FEAT), jnp.float32),
        ]

    def body(table_hbm, packed_hbm, out_hbm, *rest):
        if count:
            (cnt_hbm, packed_v, src_a, dst_a, src_b, dst_b, rows_a, rows_b,
             acc_sh, sem_a, sem_b, cnt_v, iota_v, cnt_sh) = rest
        else:
            (packed_v, src_a, dst_a, src_b, dst_b, rows_a, rows_b,
             acc_sh, sem_a, sem_b) = rest
        c = lax.axis_index("c")
        s = lax.axis_index("s")
        wid = c * 16 + s

        zerof = jnp.zeros((16,), jnp.float32)

        def zrow(r, carry):
            for j in range(D_FEAT // 16):
                rows_a[r, pl.ds(j * 16, 16)] = zerof
            return carry

        lax.fori_loop(0, CHUNK, zrow, 0)
        off = 0
        while off < rows_per_tile:
            sz = min(CHUNK, rows_per_tile - off)
            pltpu.sync_copy(
                rows_a.at[pl.ds(0, sz)],
                acc_sh.at[pl.ds(s * rows_per_tile + off, sz)],
            )
            off += sz
        if count:
            def zcnt(r, carry):
                for j in range(D_FEAT // 16):
                    cnt_v[r, pl.ds(j * 16, 16)] = zerof
                return carry

            lax.fori_loop(0, CNT_ROWS, zcnt, 0)

            @pl.when(s == 0)
            def _():
                pltpu.sync_copy(cnt_v, cnt_sh)

            def ziota(i, carry):
                iota_v[pl.ds(i * 16, 16)] = (
                    jnp.arange(16, dtype=jnp.int32) + i * 16)
                return carry

            lax.fori_loop(0, CNT_ROWS // 16, ziota, 0)
        plsc.subcore_barrier()

        ones16 = jnp.ones((16,), jnp.float32)

        def unpack(j, srcb, dstb):
            for g in range(CHUNK // 16):
                v = packed_v[j, pl.ds(g * 16, 16)]
                srcb[pl.ds(g * 16, 16)] = jnp.bitwise_and(v, 16383)
                dstb[pl.ds(g * 16, 16)] = lax.shift_right_logical(v, 14)

        def do_count(j):
            for g in range(CHUNK // 16):
                idx = lax.shift_right_logical(
                    packed_v[j, pl.ds(g * 16, 16)], 14)
                plsc.addupdate_scatter(
                    cnt_v,
                    [lax.shift_right_logical(idx, 7),
                     jnp.bitwise_and(idx, 127)],
                    ones16)

        HF, HS = K_FAST // 2, K_SLOW // 2
        for h in range(2):
            @pl.when(c == 0)
            def _():
                pltpu.sync_copy(
                    packed_hbm.at[wid].at[pl.ds(h * HF, HF)], packed_v)

            @pl.when(c == 1)
            def _():
                pltpu.sync_copy(
                    packed_hbm.at[wid].at[pl.ds(h * HS, HS)],
                    packed_v.at[pl.ds(0, HS)])

            n_chunks = jnp.where(c == 0, HF, HS)

            unpack(0, src_a, dst_a)
            pltpu.async_copy(table_hbm.at[src_a], rows_a, sem_a)
            unpack(1, src_b, dst_b)
            pltpu.async_copy(table_hbm.at[src_b], rows_b, sem_b)

            def pair(p, carry):
                j0 = 2 * p
                pltpu.make_async_copy(
                    table_hbm.at[pl.ds(0, CHUNK)], rows_a, sem_a).wait()
                pltpu.sync_copy(rows_a, acc_sh.at[dst_a], add=True)

                @pl.when(j0 + 2 < n_chunks)
                def _():
                    unpack(j0 + 2, src_a, dst_a)
                    pltpu.async_copy(table_hbm.at[src_a], rows_a, sem_a)
                if count:
                    do_count(j0)

                pltpu.make_async_copy(
                    table_hbm.at[pl.ds(0, CHUNK)], rows_b, sem_b).wait()
                pltpu.sync_copy(rows_b, acc_sh.at[dst_b], add=True)

                @pl.when(j0 + 3 < n_chunks)
                def _():
                    unpack(j0 + 3, src_b, dst_b)
                    pltpu.async_copy(table_hbm.at[src_b], rows_b, sem_b)
                if count:
                    do_count(j0 + 1)
                return carry

            lax.fori_loop(0, n_chunks // 2, pair, 0)

        if count:
            pltpu.sync_copy(cnt_v, cnt_sh.at[iota_v], add=True)
        plsc.subcore_barrier()

        pltpu.sync_copy(
            acc_sh.at[pl.ds(s * rows_per_tile, rows_per_tile)],
            out_hbm.at[c].at[pl.ds(s * rows_per_tile, rows_per_tile)],
        )
        if count:
            @pl.when(s < CNT_ROWS // 16)
            def _():
                pltpu.sync_copy(
                    cnt_sh.at[pl.ds(s * 16, 16)],
                    cnt_hbm.at[c].at[pl.ds(s * 16, 16)],
                )

    return functools.partial(
        pl.kernel, mesh=mesh, out_type=out_type, scratch_types=scratch,
        compiler_params=pltpu.CompilerParams(needs_layout_passes=False))(body)


_sc_scatter_cnt = _make_sc_scatter(count=True)
_sc_scatter = _make_sc_scatter(count=False)


def _tc1_body(p_ref, c_ref, x_ref, w1lT_ref, b1l_ref, w1rT_ref, gnw_ref,
              gnb_ref, gnms_ref, h_ref, invc_ref):
    agg = p_ref[0, :N_NODES] + p_ref[1, :N_NODES]
    cnt = c_ref[0] + c_ref[1]
    invc = 1.0 / jnp.maximum(cnt, 1.0)
    mean = agg * invc
    h = (jnp.dot(mean, w1lT_ref[...], preferred_element_type=jnp.float32)
         + b1l_ref[...]
         + jnp.dot(x_ref[...], w1rT_ref[...], preferred_element_type=jnp.float32))
    mu = jnp.mean(h, axis=0, keepdims=True)
    o = h - gnms_ref[...] * mu
    var = jnp.mean(o * o, axis=0, keepdims=True)
    g = gnw_ref[...] * o * lax.rsqrt(var + EPS) + gnb_ref[...]
    h_ref[...] = jnp.maximum(g, 0.0)
    invc_ref[...] = invc


def _tc2_body(p_ref, h_ref, invc_ref, w2lT_ref, b2l_ref, w2rT_ref, out_ref):
    mean = (p_ref[0, :N_NODES] + p_ref[1, :N_NODES]) * invc_ref[...]
    out_ref[...] = (jnp.dot(mean, w2lT_ref[...], preferred_element_type=jnp.float32)
                    + b2l_ref[...]
                    + jnp.dot(h_ref[...], w2rT_ref[...],
                              preferred_element_type=jnp.float32))


def kernel(x, edge_index, W1l, b1l, W1r, gn_w, gn_b, gn_ms, W2l, b2l, W2r):
    src = edge_index[0]
    dst = edge_index[1]
    e = src.shape[0]
    pad = E_PAD - e
    packed = jnp.concatenate(
        [jnp.bitwise_or(src, jnp.left_shift(dst, 14)),
         jnp.full((pad,), DUMMY, jnp.int32)])
    fast_part = packed[:E_FAST].reshape(16, K_FAST, CHUNK)
    slow_part = jnp.concatenate(
        [packed[E_FAST:].reshape(16, K_SLOW, CHUNK),
         jnp.full((16, K_FAST - K_SLOW, CHUNK), DUMMY, jnp.int32)], axis=1)
    packed_p = jnp.concatenate([fast_part, slow_part])

    part1, cnt_p = _sc_scatter_cnt(x, packed_p)
    cnt2 = cnt_p.reshape(2, CNT_ROWS * D_FEAT, 1)[:, :N_NODES]

    h, invc = pl.pallas_call(
        _tc1_body,
        out_shape=[
            jax.ShapeDtypeStruct((N_NODES, D_FEAT), jnp.float32),
            jax.ShapeDtypeStruct((N_NODES, 1), jnp.float32),
        ],
    )(part1, cnt2, x, W1l.T, b1l.reshape(1, -1), W1r.T, gn_w.reshape(1, -1),
      gn_b.reshape(1, -1), gn_ms.reshape(1, -1))

    (part2,) = _sc_scatter(h, packed_p)

    out = pl.pallas_call(
        _tc2_body,
        out_shape=jax.ShapeDtypeStruct((N_NODES, D_FEAT), jnp.float32),
    )(part2, h, invc, W2l.T, b2l.reshape(1, -1), W2r.T)
    return out


# speedup vs baseline: 3.8565x; 1.0002x over previous
import functools

import jax
import jax.numpy as jnp
from jax import lax
from jax.experimental import pallas as pl
from jax.experimental.pallas import tpu as pltpu
from jax.experimental.pallas import tpu_sc as plsc

N_NODES = 10000
D_FEAT = 128
EPS = 1e-5

NW = 32
CHUNK = 64
K_FAST = 240
K_SLOW = 80
E_FAST = 16 * K_FAST * CHUNK
E_SLOW = 16 * K_SLOW * CHUNK
E_PAD = E_FAST + E_SLOW
ACC_ROWS = 10112
CNT_ROWS = 80
DUMMY = 10000 << 14


def _make_sc_scatter(count: bool):
    mesh = plsc.VectorSubcoreMesh(core_axis_name="c", subcore_axis_name="s")
    rows_per_tile = ACC_ROWS // 16

    out_type = [jax.ShapeDtypeStruct((2, ACC_ROWS, D_FEAT), jnp.float32)]
    scratch = [
        pltpu.VMEM((K_FAST // 2, CHUNK), jnp.int32),
        pltpu.VMEM((CHUNK,), jnp.int32),
        pltpu.VMEM((CHUNK,), jnp.int32),
        pltpu.VMEM((CHUNK,), jnp.int32),
        pltpu.VMEM((CHUNK,), jnp.int32),
        pltpu.VMEM((CHUNK, D_FEAT), jnp.float32),
        pltpu.VMEM((CHUNK, D_FEAT), jnp.float32),
        pltpu.VMEM_SHARED((ACC_ROWS, D_FEAT), jnp.float32),
        pltpu.SemaphoreType.DMA,
        pltpu.SemaphoreType.DMA,
    ]
    if count:
        out_type.append(jax.ShapeDtypeStruct((2, CNT_ROWS, D_FEAT), jnp.float32))
        scratch += [
            pltpu.VMEM((CNT_ROWS, D_FEAT), jnp.float32),
            pltpu.VMEM((CNT_ROWS,), jnp.int32),
            pltpu.VMEM_SHARED((CNT_ROWS, D_FEAT), jnp.float32),
        ]

    def body(table_hbm, packed_hbm, out_hbm, *rest):
        if count:
            (cnt_hbm, packed_v, src_a, dst_a, src_b, dst_b, rows_a, rows_b,
             acc_sh, sem_a, sem_b, cnt_v, iota_v, cnt_sh) = rest
        else:
            (packed_v, src_a, dst_a, src_b, dst_b, rows_a, rows_b,
             acc_sh, sem_a, sem_b) = rest
        c = lax.axis_index("c")
        s = lax.axis_index("s")
        wid = c * 16 + s

        zerof = jnp.zeros((16,), jnp.float32)

        def zero_phase():
            def zrow(r, carry):
                for j in range(D_FEAT // 16):
                    rows_a[r, pl.ds(j * 16, 16)] = zerof
                return carry

            lax.fori_loop(0, CHUNK, zrow, 0)
            off = 0
            while off < rows_per_tile:
                sz = min(CHUNK, rows_per_tile - off)
                pltpu.sync_copy(
                    rows_a.at[pl.ds(0, sz)],
                    acc_sh.at[pl.ds(s * rows_per_tile + off, sz)],
                )
                off += sz
            if count:
                def zcnt(r, carry):
                    for j in range(D_FEAT // 16):
                        cnt_v[r, pl.ds(j * 16, 16)] = zerof
                    return carry

                lax.fori_loop(0, CNT_ROWS, zcnt, 0)

                @pl.when(s == 0)
                def _():
                    pltpu.sync_copy(cnt_v, cnt_sh)

                def ziota(i, carry):
                    iota_v[pl.ds(i * 16, 16)] = (
                        jnp.arange(16, dtype=jnp.int32) + i * 16)
                    return carry

                lax.fori_loop(0, CNT_ROWS // 16, ziota, 0)
            plsc.subcore_barrier()

        with jax.named_scope("ph_zero"):
            zero_phase()

        ones16 = jnp.ones((16,), jnp.float32)

        def unpack(j, srcb, dstb):
            for g in range(CHUNK // 16):
                v = packed_v[j, pl.ds(g * 16, 16)]
                srcb[pl.ds(g * 16, 16)] = jnp.bitwise_and(v, 16383)
                dstb[pl.ds(g * 16, 16)] = lax.shift_right_logical(v, 14)

        def do_count(j):
            for g in range(CHUNK // 16):
                idx = lax.shift_right_logical(
                    packed_v[j, pl.ds(g * 16, 16)], 14)
                plsc.addupdate_scatter(
                    cnt_v,
                    [lax.shift_right_logical(idx, 7),
                     jnp.bitwise_and(idx, 127)],
                    ones16)

        HF, HS = K_FAST // 2, K_SLOW // 2

        def run_half(h):
            @pl.when(c == 0)
            def _():
                pltpu.sync_copy(
                    packed_hbm.at[wid].at[pl.ds(h * HF, HF)], packed_v)

            @pl.when(c == 1)
            def _():
                pltpu.sync_copy(
                    packed_hbm.at[wid].at[pl.ds(h * HS, HS)],
                    packed_v.at[pl.ds(0, HS)])

            n_chunks = jnp.where(c == 0, HF, HS)

            unpack(0, src_a, dst_a)
            pltpu.async_copy(table_hbm.at[src_a], rows_a, sem_a)
            unpack(1, src_b, dst_b)
            pltpu.async_copy(table_hbm.at[src_b], rows_b, sem_b)

            def pair(p, carry):
                j0 = 2 * p
                pltpu.make_async_copy(
                    table_hbm.at[pl.ds(0, CHUNK)], rows_a, sem_a).wait()
                pltpu.sync_copy(rows_a, acc_sh.at[dst_a], add=True)

                @pl.when(j0 + 2 < n_chunks)
                def _():
                    unpack(j0 + 2, src_a, dst_a)
                    pltpu.async_copy(table_hbm.at[src_a], rows_a, sem_a)
                if count:
                    do_count(j0)

                pltpu.make_async_copy(
                    table_hbm.at[pl.ds(0, CHUNK)], rows_b, sem_b).wait()
                pltpu.sync_copy(rows_b, acc_sh.at[dst_b], add=True)

                @pl.when(j0 + 3 < n_chunks)
                def _():
                    unpack(j0 + 3, src_b, dst_b)
                    pltpu.async_copy(table_hbm.at[src_b], rows_b, sem_b)
                if count:
                    do_count(j0 + 1)
                return carry

            lax.fori_loop(0, n_chunks // 2, pair, 0)

        for h in range(2):
            with jax.named_scope(f"ph_half{h}"):
                run_half(h)

        def tail_phase():
            if count:
                pltpu.sync_copy(cnt_v, cnt_sh.at[iota_v], add=True)
            plsc.subcore_barrier()

            pltpu.sync_copy(
                acc_sh.at[pl.ds(s * rows_per_tile, rows_per_tile)],
                out_hbm.at[c].at[pl.ds(s * rows_per_tile, rows_per_tile)],
            )
            if count:
                @pl.when(s < CNT_ROWS // 16)
                def _():
                    pltpu.sync_copy(
                        cnt_sh.at[pl.ds(s * 16, 16)],
                        cnt_hbm.at[c].at[pl.ds(s * 16, 16)],
                    )

        with jax.named_scope("ph_tail"):
            tail_phase()

    return functools.partial(
        pl.kernel, mesh=mesh, out_type=out_type, scratch_types=scratch,
        compiler_params=pltpu.CompilerParams(needs_layout_passes=False))(body)


_sc_scatter_cnt = _make_sc_scatter(count=True)
_sc_scatter = _make_sc_scatter(count=False)


def _tc1_body(p_ref, c_ref, x_ref, w1lT_ref, b1l_ref, w1rT_ref, gnw_ref,
              gnb_ref, gnms_ref, h_ref, invc_ref):
    agg = p_ref[0, :N_NODES] + p_ref[1, :N_NODES]
    cnt = c_ref[0] + c_ref[1]
    invc = 1.0 / jnp.maximum(cnt, 1.0)
    mean = agg * invc
    h = (jnp.dot(mean, w1lT_ref[...], preferred_element_type=jnp.float32)
         + b1l_ref[...]
         + jnp.dot(x_ref[...], w1rT_ref[...], preferred_element_type=jnp.float32))
    mu = jnp.mean(h, axis=0, keepdims=True)
    o = h - gnms_ref[...] * mu
    var = jnp.mean(o * o, axis=0, keepdims=True)
    g = gnw_ref[...] * o * lax.rsqrt(var + EPS) + gnb_ref[...]
    h_ref[...] = jnp.maximum(g, 0.0)
    invc_ref[...] = invc


def _tc2_body(p_ref, h_ref, invc_ref, w2lT_ref, b2l_ref, w2rT_ref, out_ref):
    mean = (p_ref[0, :N_NODES] + p_ref[1, :N_NODES]) * invc_ref[...]
    out_ref[...] = (jnp.dot(mean, w2lT_ref[...], preferred_element_type=jnp.float32)
                    + b2l_ref[...]
                    + jnp.dot(h_ref[...], w2rT_ref[...],
                              preferred_element_type=jnp.float32))


def kernel(x, edge_index, W1l, b1l, W1r, gn_w, gn_b, gn_ms, W2l, b2l, W2r):
    src = edge_index[0]
    dst = edge_index[1]
    e = src.shape[0]
    pad = E_PAD - e
    packed = jnp.concatenate(
        [jnp.bitwise_or(src, jnp.left_shift(dst, 14)),
         jnp.full((pad,), DUMMY, jnp.int32)])
    fast_part = packed[:E_FAST].reshape(16, K_FAST, CHUNK)
    slow_part = jnp.concatenate(
        [packed[E_FAST:].reshape(16, K_SLOW, CHUNK),
         jnp.full((16, K_FAST - K_SLOW, CHUNK), DUMMY, jnp.int32)], axis=1)
    packed_p = jnp.concatenate([fast_part, slow_part])

    part1, cnt_p = _sc_scatter_cnt(x, packed_p)
    cnt2 = cnt_p.reshape(2, CNT_ROWS * D_FEAT, 1)[:, :N_NODES]

    h, invc = pl.pallas_call(
        _tc1_body,
        out_shape=[
            jax.ShapeDtypeStruct((N_NODES, D_FEAT), jnp.float32),
            jax.ShapeDtypeStruct((N_NODES, 1), jnp.float32),
        ],
    )(part1, cnt2, x, W1l.T, b1l.reshape(1, -1), W1r.T, gn_w.reshape(1, -1),
      gn_b.reshape(1, -1), gn_ms.reshape(1, -1))

    (part2,) = _sc_scatter(h, packed_p)

    out = pl.pallas_call(
        _tc2_body,
        out_shape=jax.ShapeDtypeStruct((N_NODES, D_FEAT), jnp.float32),
    )(part2, h, invc, W2l.T, b2l.reshape(1, -1), W2r.T)
    return out
